# Initial kernel scaffold; baseline (speedup 1.0000x reference)
import jax
import jax.numpy as jnp
from jax.experimental import pallas as pl


def kernel(positions, table):
    raise NotImplementedError("write your pallas kernel here")


# speedup vs baseline: 4.1554x; 4.1554x over previous
import functools

import jax
import jax.numpy as jnp
from jax import lax
from jax.experimental import pallas as pl
from jax.experimental.pallas import tpu as pltpu
from jax.experimental.pallas import tpu_sc as plsc

MAX_LEN = 8192
EMB_DIM = 64
N_ROWS = 4096
N_COLS = 200

NC = 2
NS = 16
NW = NC * NS

B = N_ROWS * N_COLS
B_PER_W = B // NW
CHUNK = 128
N_CHUNKS = B_PER_W // CHUNK


def _body(idx_hbm, table_hbm, out_hbm, idx_v, rows_v, gsem):
    wid = lax.axis_index("s") * NC + lax.axis_index("c")
    pltpu.sync_copy(idx_hbm.at[wid], idx_v)

    def step(j, carry):
        pltpu.async_copy(table_hbm.at[idx_v.at[j]], rows_v, gsem).wait()
        pltpu.sync_copy(rows_v, out_hbm.at[wid, j])
        return carry

    lax.fori_loop(0, N_CHUNKS, step, 0)


@functools.partial(jax.jit, static_argnums=())
def _gather(positions_flat, table):
    mesh = plsc.VectorSubcoreMesh(core_axis_name="c", subcore_axis_name="s")
    idx = positions_flat.reshape(NW, N_CHUNKS, CHUNK)
    out = pl.kernel(
        _body,
        out_type=jax.ShapeDtypeStruct((NW, N_CHUNKS, CHUNK, EMB_DIM), jnp.float32),
        mesh=mesh,
        scratch_types=[
            pltpu.VMEM((N_CHUNKS, CHUNK), jnp.int32),
            pltpu.VMEM((CHUNK, EMB_DIM), jnp.float32),
            pltpu.SemaphoreType.DMA,
        ],
        compiler_params=pltpu.CompilerParams(use_tc_tiling_on_sc=False),
    )(idx, table)
    return out


def kernel(positions, table):
    out = _gather(positions.reshape(-1), table)
    return out.reshape(N_ROWS, N_COLS, EMB_DIM)


# speedup vs baseline: 4.7503x; 1.1432x over previous
import functools

import jax
import jax.numpy as jnp
from jax import lax
from jax.experimental import pallas as pl
from jax.experimental.pallas import tpu as pltpu
from jax.experimental.pallas import tpu_sc as plsc

MAX_LEN = 8192
EMB_DIM = 64
N_ROWS = 4096
N_COLS = 200

NC = 2
NS = 16
NW = NC * NS

B = N_ROWS * N_COLS
B_PER_W = B // NW
CHUNK = 128
N_CHUNKS = B_PER_W // CHUNK


def _body(idx_hbm, table_hbm, out_hbm, idx_v, rows_v, ga, gb, oa, ob):
    wid = lax.axis_index("s") * NC + lax.axis_index("c")
    pltpu.sync_copy(idx_hbm.at[wid], idx_v)

    buf_a, buf_b = rows_v.at[0], rows_v.at[1]

    def gather(j, buf, sem):
        pltpu.async_copy(table_hbm.at[idx_v.at[j]], buf, sem)

    def gather_wait(buf, sem):
        pltpu.make_async_copy(table_hbm.at[idx_v.at[0]], buf, sem).wait()

    def write(j, buf, sem):
        pltpu.async_copy(buf, out_hbm.at[wid, j], sem)

    def write_wait(buf, sem):
        pltpu.make_async_copy(buf, out_hbm.at[wid, 0], sem).wait()

    gather(0, buf_a, ga)
    gather(1, buf_b, gb)

    def step(jj, carry):
        j0 = jj * 2
        gather_wait(buf_a, ga)
        write(j0, buf_a, oa)
        gather_wait(buf_b, gb)
        write(j0 + 1, buf_b, ob)

        @pl.when(jj < N_CHUNKS // 2 - 1)
        def _():
            write_wait(buf_a, oa)
            gather(j0 + 2, buf_a, ga)
            write_wait(buf_b, ob)
            gather(j0 + 3, buf_b, gb)

        return carry

    lax.fori_loop(0, N_CHUNKS // 2, step, 0)
    write_wait(buf_a, oa)
    write_wait(buf_b, ob)


@functools.partial(jax.jit, static_argnums=())
def _gather(positions_flat, table):
    mesh = plsc.VectorSubcoreMesh(core_axis_name="c", subcore_axis_name="s")
    idx = positions_flat.reshape(NW, N_CHUNKS, CHUNK)
    out = pl.kernel(
        _body,
        out_type=jax.ShapeDtypeStruct((NW, N_CHUNKS, CHUNK, EMB_DIM), jnp.float32),
        mesh=mesh,
        scratch_types=[
            pltpu.VMEM((N_CHUNKS, CHUNK), jnp.int32),
            pltpu.VMEM((2, CHUNK, EMB_DIM), jnp.float32),
            pltpu.SemaphoreType.DMA,
            pltpu.SemaphoreType.DMA,
            pltpu.SemaphoreType.DMA,
            pltpu.SemaphoreType.DMA,
        ],
        compiler_params=pltpu.CompilerParams(use_tc_tiling_on_sc=False),
    )(idx, table)
    return out


def kernel(positions, table):
    out = _gather(positions.reshape(-1), table)
    return out.reshape(N_ROWS, N_COLS, EMB_DIM)


# speedup vs baseline: 5.1649x; 1.0873x over previous
import functools

import jax
import jax.numpy as jnp
from jax import lax
from jax.experimental import pallas as pl
from jax.experimental.pallas import tpu as pltpu
from jax.experimental.pallas import tpu_sc as plsc

MAX_LEN = 8192
EMB_DIM = 64
N_ROWS = 4096
N_COLS = 200

NC = 2
NS = 16
NW = NC * NS

B = N_ROWS * N_COLS
B_PER_W = B // NW
CHUNK = 128
N_CHUNKS = B_PER_W // CHUNK


def _body(idx_hbm, table_hbm, out_hbm, idx_v, rows_v, table_sh, ga, gb, oa, ob):
    wid = lax.axis_index("s") * NC + lax.axis_index("c")
    pltpu.sync_copy(idx_hbm.at[wid], idx_v)

    @pl.when(lax.axis_index("s") == 0)
    def _():
        pltpu.sync_copy(table_hbm, table_sh)

    plsc.subcore_barrier()

    buf_a, buf_b = rows_v.at[0], rows_v.at[1]

    def gather(j, buf, sem):
        pltpu.async_copy(table_sh.at[idx_v.at[j]], buf, sem)

    def gather_wait(buf, sem):
        pltpu.make_async_copy(table_sh.at[idx_v.at[0]], buf, sem).wait()

    def write(j, buf, sem):
        pltpu.async_copy(buf, out_hbm.at[wid, j], sem)

    def write_wait(buf, sem):
        pltpu.make_async_copy(buf, out_hbm.at[wid, 0], sem).wait()

    gather(0, buf_a, ga)
    gather(1, buf_b, gb)

    def step(jj, carry):
        j0 = jj * 2
        gather_wait(buf_a, ga)
        write(j0, buf_a, oa)
        gather_wait(buf_b, gb)
        write(j0 + 1, buf_b, ob)

        @pl.when(jj < N_CHUNKS // 2 - 1)
        def _():
            write_wait(buf_a, oa)
            gather(j0 + 2, buf_a, ga)
            write_wait(buf_b, ob)
            gather(j0 + 3, buf_b, gb)

        return carry

    lax.fori_loop(0, N_CHUNKS // 2, step, 0)
    write_wait(buf_a, oa)
    write_wait(buf_b, ob)


@functools.partial(jax.jit, static_argnums=())
def _gather(positions_flat, table):
    mesh = plsc.VectorSubcoreMesh(core_axis_name="c", subcore_axis_name="s")
    idx = positions_flat.reshape(NW, N_CHUNKS, CHUNK)
    out = pl.kernel(
        _body,
        out_type=jax.ShapeDtypeStruct((NW, N_CHUNKS, CHUNK, EMB_DIM), jnp.float32),
        mesh=mesh,
        scratch_types=[
            pltpu.VMEM((N_CHUNKS, CHUNK), jnp.int32),
            pltpu.VMEM((2, CHUNK, EMB_DIM), jnp.float32),
            pltpu.VMEM_SHARED((MAX_LEN, EMB_DIM), jnp.float32),
            pltpu.SemaphoreType.DMA,
            pltpu.SemaphoreType.DMA,
            pltpu.SemaphoreType.DMA,
            pltpu.SemaphoreType.DMA,
        ],
        compiler_params=pltpu.CompilerParams(use_tc_tiling_on_sc=False),
    )(idx, table)
    return out


def kernel(positions, table):
    out = _gather(positions.reshape(-1), table)
    return out.reshape(N_ROWS, N_COLS, EMB_DIM)


# speedup vs baseline: 5.5758x; 1.0796x over previous
import jax
import jax.numpy as jnp
from jax import lax
from jax.experimental import pallas as pl
from jax.experimental.pallas import tpu as pltpu
from jax.experimental.pallas import tpu_sc as plsc

MAX_LEN = 8192
EMB_DIM = 64
N_ROWS = 4096
N_COLS = 200
HALF = N_COLS // 2

NC = 2
NS = 16
NW = NC * NS
ROWS_PER_W = N_ROWS // NW


def _body(idx_hbm, table_hbm, out_hbm, idx_v, rows_v, table_sh, ga, gb, oa, ob):
    wid = lax.axis_index("s") * NC + lax.axis_index("c")
    pltpu.sync_copy(idx_hbm.at[wid], idx_v)

    @pl.when(lax.axis_index("s") == 0)
    def _():
        pltpu.sync_copy(table_hbm, table_sh)

    plsc.subcore_barrier()

    buf_a, buf_b = rows_v.at[0], rows_v.at[1]

    def gather(r, buf):
        pltpu.async_copy(table_sh.at[idx_v.at[2 * r]], buf.at[pl.ds(0, HALF)], ga)
        pltpu.async_copy(table_sh.at[idx_v.at[2 * r + 1]], buf.at[pl.ds(HALF, HALF)], gb)

    def gather_wait(buf):
        pltpu.make_async_copy(table_sh.at[idx_v.at[0]], buf.at[pl.ds(0, HALF)], ga).wait()
        pltpu.make_async_copy(table_sh.at[idx_v.at[0]], buf.at[pl.ds(HALF, HALF)], gb).wait()

    def write(r, buf, sem):
        pltpu.async_copy(buf, out_hbm.at[wid * ROWS_PER_W + r], sem)

    def write_wait(buf, sem):
        pltpu.make_async_copy(buf, out_hbm.at[0], sem).wait()

    gather(0, buf_a)
    gather(1, buf_b)

    def step(rr, carry):
        r0 = rr * 2
        gather_wait(buf_a)
        write(r0, buf_a, oa)
        gather_wait(buf_b)
        write(r0 + 1, buf_b, ob)

        @pl.when(rr < ROWS_PER_W // 2 - 1)
        def _():
            write_wait(buf_a, oa)
            gather(r0 + 2, buf_a)
            write_wait(buf_b, ob)
            gather(r0 + 3, buf_b)

        return carry

    lax.fori_loop(0, ROWS_PER_W // 2, step, 0)
    write_wait(buf_a, oa)
    write_wait(buf_b, ob)


@jax.jit
def _gather_op(positions, table):
    mesh = plsc.VectorSubcoreMesh(core_axis_name="c", subcore_axis_name="s")
    idx = positions.reshape(NW, ROWS_PER_W * 2, HALF)
    out = pl.kernel(
        _body,
        out_type=jax.ShapeDtypeStruct((N_ROWS, N_COLS, EMB_DIM), jnp.float32),
        mesh=mesh,
        scratch_types=[
            pltpu.VMEM((ROWS_PER_W * 2, HALF), jnp.int32),
            pltpu.VMEM((2, N_COLS, EMB_DIM), jnp.float32),
            pltpu.VMEM_SHARED((MAX_LEN, EMB_DIM), jnp.float32),
            pltpu.SemaphoreType.DMA,
            pltpu.SemaphoreType.DMA,
            pltpu.SemaphoreType.DMA,
            pltpu.SemaphoreType.DMA,
        ],
        compiler_params=pltpu.CompilerParams(use_tc_tiling_on_sc=False),
    )(idx, table)
    return out


def kernel(positions, table):
    return _gather_op(positions, table)
